# baseline (speedup 1.0000x reference)
import functools

import jax
import jax.numpy as jnp
from jax.experimental import pallas as pl
from jax.experimental.pallas import tpu as pltpu


def _layernorm(x, g, b, eps=1e-5):
    m = jnp.mean(x, axis=-1, keepdims=True)
    xc = x - m
    v = jnp.mean(xc * xc, axis=-1, keepdims=True)
    return xc * jax.lax.rsqrt(v + eps) * g + b


def _fused_block_kernel(adjt_ref, adjb_ref, xsrc_ref, xdst_ref, wrel_ref,
                        wn_ref, ws_ref, bcat_ref, wff_ref, bff_ref, ghn_ref,
                        bhn_ref, gfn_ref, bfn_ref, out_ref):
    xsrc = xsrc_ref[...]
    acc = jnp.concatenate(
        [jnp.dot(adjt_ref[...].astype(jnp.bfloat16), xsrc,
                 preferred_element_type=jnp.float32),
         jnp.dot(adjb_ref[...].astype(jnp.bfloat16), xsrc,
                 preferred_element_type=jnp.float32)], axis=0)
    nb = jnp.dot(acc, wrel_ref[...], preferred_element_type=jnp.float32)
    x = xdst_ref[...]
    out = (jnp.dot(nb, wn_ref[...], preferred_element_type=jnp.float32)
           + jnp.dot(x, ws_ref[...], preferred_element_type=jnp.float32)
           + bcat_ref[...])
    y = _layernorm(out + x, ghn_ref[...], bhn_ref[...])
    z = jax.nn.relu(jnp.dot(y, wff_ref[...], preferred_element_type=jnp.float32)
                    + bff_ref[...])
    out_ref[...] = _layernorm(z + y, gfn_ref[...], bfn_ref[...])


@functools.partial(jax.jit, static_argnames=("block_rows",))
def _hete_block(adj, x_src_bf16, x_dst, w_rel, wn, ws, bcat, wff, bff,
                g_hn, b_hn, g_fn, b_fn, block_rows=80):
    m, n = adj.shape
    d = x_dst.shape[1]
    br = min(block_rows, m)
    brh = br // 2
    grid = (pl.cdiv(m, br),)
    row2 = lambda i: (i, 0)
    top = lambda i: (2 * i, 0)
    bot = lambda i: (2 * i + 1, 0)
    full = lambda i: (0, 0)
    vec_spec = pl.BlockSpec((1, d), full)
    mat_spec = pl.BlockSpec((d, d), full)
    return pl.pallas_call(
        _fused_block_kernel,
        grid=grid,
        in_specs=[
            pl.BlockSpec((brh, n), top),
            pl.BlockSpec((brh, n), bot),
            pl.BlockSpec((n, d), full),
            pl.BlockSpec((br, d), row2),
            mat_spec, mat_spec, mat_spec,
            vec_spec,
            mat_spec, vec_spec,
            vec_spec, vec_spec, vec_spec, vec_spec,
        ],
        out_specs=pl.BlockSpec((br, d), row2),
        out_shape=jax.ShapeDtypeStruct((m, d), jnp.float32),
        compiler_params=pltpu.CompilerParams(
            dimension_semantics=("arbitrary",)),
    )(adj, adj, x_src_bf16, x_dst, w_rel, wn, ws, bcat, wff, bff,
      g_hn, b_hn, g_fn, b_fn)


def kernel(x_p, x_a, adj_p_a, adj_a_p, W_rel_p_a, W_rel_a_p, Wcat_p, bcat_p,
           Wcat_a, bcat_a, Wff_p, bff_p, Wff_a, bff_a, g_hn_p, g_hn_a,
           g_fn_p, g_fn_a, b_hn_p, b_hn_a, b_fn_p, b_fn_a):
    d = x_p.shape[1]
    row = lambda v: v.reshape(1, d)
    z_p = _hete_block(
        adj_p_a, x_a.astype(jnp.bfloat16), x_p, W_rel_p_a,
        Wcat_p[:, :d].T, Wcat_p[:, d:].T, row(bcat_p),
        Wff_p.T, row(bff_p), row(g_hn_p), row(b_hn_p), row(g_fn_p),
        row(b_fn_p))
    z_a = _hete_block(
        adj_a_p, x_p.astype(jnp.bfloat16), x_a, W_rel_a_p,
        Wcat_a[:, :d].T, Wcat_a[:, d:].T, row(bcat_a),
        Wff_a.T, row(bff_a), row(g_hn_a), row(b_hn_a), row(g_fn_a),
        row(b_fn_a))
    return (z_p, z_a)


# speedup vs baseline: 1.7840x; 1.7840x over previous
import functools

import jax
import jax.numpy as jnp
from jax.experimental import pallas as pl
from jax.experimental.pallas import tpu as pltpu

_NBUF = 3
_LOOK = 2
_OBUF = 2


def _layernorm(x, g, b, eps=1e-5):
    m = jnp.mean(x, axis=-1, keepdims=True)
    xc = x - m
    v = jnp.mean(xc * xc, axis=-1, keepdims=True)
    return xc * jax.lax.rsqrt(v + eps) * g + b


def _mega_kernel(br, nblk,
                 adjp_hbm, adja_hbm, xp_hbm, xa_hbm, xab_ref, xpb_ref,
                 wrel_p, wn_p, ws_p, bcat_p, wff_p, bff_p,
                 ghn_p, bhn_p, gfn_p, bfn_p,
                 wrel_a, wn_a, ws_a, bcat_a, wff_a, bff_a,
                 ghn_a, bhn_a, gfn_a, bfn_a,
                 outp_hbm, outa_hbm,
                 adj_ring, xd_ring, ob_ring, adj_sem, xd_sem, ob_sem):
    tot = 2 * nblk

    def start_fetch(g):
        g = jnp.int32(g)
        slot = jax.lax.rem(g, _NBUF)

        @pl.when(g < nblk)
        def _():
            rows = pl.ds(g * br, br)
            pltpu.make_async_copy(adjp_hbm.at[rows, :], adj_ring.at[slot],
                                  adj_sem.at[slot]).start()
            pltpu.make_async_copy(xp_hbm.at[rows, :], xd_ring.at[slot],
                                  xd_sem.at[slot]).start()

        @pl.when(jnp.logical_and(g >= nblk, g < tot))
        def _():
            rows = pl.ds((g - nblk) * br, br)
            pltpu.make_async_copy(adja_hbm.at[rows, :], adj_ring.at[slot],
                                  adj_sem.at[slot]).start()
            pltpu.make_async_copy(xa_hbm.at[rows, :], xd_ring.at[slot],
                                  xd_sem.at[slot]).start()

    start_fetch(0)
    start_fetch(1)

    def run_relation(rel, xsrc_ref, adj_hbm, xd_hbm, out_hbm,
                     wrel, wn, ws, bcat, wff, bff, ghn, bhn, gfn, bfn):
        xsrc = xsrc_ref[...]

        def body(i, carry):
            g = rel * nblk + i
            start_fetch(g + _LOOK)
            slot = jax.lax.rem(g, _NBUF)
            rows = pl.ds(i * br, br)
            pltpu.make_async_copy(adj_hbm.at[rows, :], adj_ring.at[slot],
                                  adj_sem.at[slot]).wait()
            pltpu.make_async_copy(xd_hbm.at[rows, :], xd_ring.at[slot],
                                  xd_sem.at[slot]).wait()
            acc = jnp.dot(adj_ring[slot].astype(jnp.bfloat16), xsrc,
                          preferred_element_type=jnp.float32)
            nb = jnp.dot(acc, wrel[...], preferred_element_type=jnp.float32)
            x = xd_ring[slot]
            out = (jnp.dot(nb, wn[...], preferred_element_type=jnp.float32)
                   + jnp.dot(x, ws[...], preferred_element_type=jnp.float32)
                   + bcat[...])
            y = _layernorm(out + x, ghn[...], bhn[...])
            z = jax.nn.relu(
                jnp.dot(y, wff[...], preferred_element_type=jnp.float32)
                + bff[...])
            z = _layernorm(z + y, gfn[...], bfn[...])
            oslot = jax.lax.rem(g, _OBUF)

            @pl.when(g >= _OBUF)
            def _():
                pltpu.make_async_copy(ob_ring.at[oslot],
                                      out_hbm.at[pl.ds(0, br), :],
                                      ob_sem.at[oslot]).wait()

            ob_ring[oslot] = z
            pltpu.make_async_copy(ob_ring.at[oslot], out_hbm.at[rows, :],
                                  ob_sem.at[oslot]).start()
            return carry

        jax.lax.fori_loop(0, nblk, body, 0)

    run_relation(0, xab_ref, adjp_hbm, xp_hbm, outp_hbm,
                 wrel_p, wn_p, ws_p, bcat_p, wff_p, bff_p,
                 ghn_p, bhn_p, gfn_p, bfn_p)
    run_relation(1, xpb_ref, adja_hbm, xa_hbm, outa_hbm,
                 wrel_a, wn_a, ws_a, bcat_a, wff_a, bff_a,
                 ghn_a, bhn_a, gfn_a, bfn_a)

    for g in (tot - 2, tot - 1):
        pltpu.make_async_copy(ob_ring.at[g % _OBUF],
                              outa_hbm.at[pl.ds(0, br), :],
                              ob_sem.at[g % _OBUF]).wait()


@jax.jit
def _hete_layer(adj_p, adj_a, x_p, x_a, x_a_bf16, x_p_bf16,
                wrel_p, wn_p, ws_p, bcat_p, wff_p, bff_p,
                ghn_p, bhn_p, gfn_p, bfn_p,
                wrel_a, wn_a, ws_a, bcat_a, wff_a, bff_a,
                ghn_a, bhn_a, gfn_a, bfn_a):
    m, n = adj_p.shape
    d = x_p.shape[1]
    br = 400 if m % 400 == 0 else m
    nblk = m // br
    any_spec = pl.BlockSpec(memory_space=pl.ANY)
    vmem = pl.BlockSpec(memory_space=pltpu.MemorySpace.VMEM)
    n_vmem_small = 22
    return pl.pallas_call(
        functools.partial(_mega_kernel, br, nblk),
        in_specs=[any_spec, any_spec, any_spec, any_spec]
        + [vmem] * n_vmem_small,
        out_specs=(any_spec, any_spec),
        out_shape=(jax.ShapeDtypeStruct((m, d), jnp.float32),
                   jax.ShapeDtypeStruct((m, d), jnp.float32)),
        scratch_shapes=[
            pltpu.VMEM((_NBUF, br, n), jnp.float32),
            pltpu.VMEM((_NBUF, br, d), jnp.float32),
            pltpu.VMEM((_OBUF, br, d), jnp.float32),
            pltpu.SemaphoreType.DMA((_NBUF,)),
            pltpu.SemaphoreType.DMA((_NBUF,)),
            pltpu.SemaphoreType.DMA((_OBUF,)),
        ],
    )(adj_p, adj_a, x_p, x_a, x_a_bf16, x_p_bf16,
      wrel_p, wn_p, ws_p, bcat_p, wff_p, bff_p,
      ghn_p, bhn_p, gfn_p, bfn_p,
      wrel_a, wn_a, ws_a, bcat_a, wff_a, bff_a,
      ghn_a, bhn_a, gfn_a, bfn_a)


def kernel(x_p, x_a, adj_p_a, adj_a_p, W_rel_p_a, W_rel_a_p, Wcat_p, bcat_p,
           Wcat_a, bcat_a, Wff_p, bff_p, Wff_a, bff_a, g_hn_p, g_hn_a,
           g_fn_p, g_fn_a, b_hn_p, b_hn_a, b_fn_p, b_fn_a):
    d = x_p.shape[1]
    row = lambda v: v.reshape(1, d)
    return _hete_layer(
        adj_p_a, adj_a_p, x_p, x_a,
        x_a.astype(jnp.bfloat16), x_p.astype(jnp.bfloat16),
        W_rel_p_a, Wcat_p[:, :d].T, Wcat_p[:, d:].T, row(bcat_p),
        Wff_p.T, row(bff_p), row(g_hn_p), row(b_hn_p), row(g_fn_p),
        row(b_fn_p),
        W_rel_a_p, Wcat_a[:, :d].T, Wcat_a[:, d:].T, row(bcat_a),
        Wff_a.T, row(bff_a), row(g_hn_a), row(b_hn_a), row(g_fn_a),
        row(b_fn_a))
